# Initial kernel scaffold; baseline (speedup 1.0000x reference)
import jax
import jax.numpy as jnp
from jax.experimental import pallas as pl


def kernel(x, edge_index, batch, params):
    raise NotImplementedError("write your pallas kernel here")


# speedup vs baseline: 1.9934x; 1.9934x over previous
import functools

import jax
import jax.numpy as jnp
from jax import lax
from jax.experimental import pallas as pl
from jax.experimental.pallas import tpu as pltpu
from jax.experimental.pallas import tpu_sc as plsc

_N = 10000
_NPAD = 10240
_E = 160000
_EPAD = 163840
_EROWS = _EPAD // 128
_TN = 400
_KCH = 8
_HID = 1024
_DOUT = 512


@functools.lru_cache(maxsize=None)
def _seg_sum_kernel(C):
    bpc = C // 2
    rows_per_tile = _EROWS // 16
    nchunk = rows_per_tile // _KCH
    mesh = plsc.VectorSubcoreMesh(core_axis_name="c", subcore_axis_name="s")
    out_type = tuple(
        jax.ShapeDtypeStruct((_NPAD, 128), jnp.float32) for _ in range(C))

    @functools.partial(
        pl.kernel,
        out_type=out_type,
        mesh=mesh,
        scratch_types=[
            pltpu.VMEM((_KCH, 128), jnp.int32),
            pltpu.VMEM((_KCH, 128), jnp.int32),
            pltpu.VMEM((128, 128), jnp.float32),
            pltpu.VMEM_SHARED((_NPAD, 128), jnp.float32),
        ],
    )
    def k(*refs):
        tables = refs[:C]
        src2d = refs[C]
        dst2d = refs[C + 1]
        zeros = refs[C + 2]
        outs = refs[C + 3:C + 3 + C]
        src_v, dst_v, rows_v, acc = refs[C + 3 + C:]
        cid = lax.axis_index("c")
        sid = lax.axis_index("s")
        row0 = sid * (_NPAD // 16)

        def process(table):
            def chunk(kk, carry):
                base = sid * rows_per_tile + kk * _KCH
                pltpu.sync_copy(src2d.at[pl.ds(base, _KCH)], src_v)
                pltpu.sync_copy(dst2d.at[pl.ds(base, _KCH)], dst_v)
                for j in range(_KCH):
                    pltpu.sync_copy(table.at[src_v.at[j]], rows_v)
                    pltpu.sync_copy(rows_v, acc.at[dst_v.at[j]], add=True)
                return carry
            lax.fori_loop(0, nchunk, chunk, 0)

        for lb in range(bpc):
            pltpu.sync_copy(zeros.at[pl.ds(row0, _NPAD // 16)],
                            acc.at[pl.ds(row0, _NPAD // 16)])
            plsc.subcore_barrier()

            @pl.when(cid == 0)
            def _():
                process(tables[lb])

            @pl.when(cid == 1)
            def _():
                process(tables[bpc + lb])

            plsc.subcore_barrier()

            @pl.when(cid == 0)
            def _():
                pltpu.sync_copy(acc.at[pl.ds(row0, _NPAD // 16)],
                                outs[lb].at[pl.ds(row0, _NPAD // 16)])

            @pl.when(cid == 1)
            def _():
                pltpu.sync_copy(acc.at[pl.ds(row0, _NPAD // 16)],
                                outs[bpc + lb].at[pl.ds(row0, _NPAD // 16)])

            if lb + 1 < bpc:
                plsc.subcore_barrier()

    return k


def _dot(a, b):
    return lax.dot_general(a, b, (((1,), (0,)), ((), ())),
                           preferred_element_type=jnp.float32)


@functools.lru_cache(maxsize=None)
def _mm1_kernel(C_in, interpret=False):
    din = C_in * 128
    grid = (_N // _TN,)

    def body(*refs):
        eps_r = refs[0]
        aggs = refs[1:1 + C_in]
        hs = refs[1 + C_in:1 + 2 * C_in]
        w1 = refs[1 + 2 * C_in]
        b1 = refs[2 + 2 * C_in]
        h1_o, ssum_o, ssq_o = refs[3 + 2 * C_in:]
        i = pl.program_id(0)
        s = eps_r[0, 0]
        xs = [aggs[c][...] + s * hs[c][...] for c in range(C_in)]
        x = jnp.concatenate(xs, axis=1) if C_in > 1 else xs[0]
        h1 = _dot(x, w1[...]) + b1[...]
        h1_o[...] = h1

        @pl.when(i == 0)
        def _():
            ssum_o[...] = jnp.zeros_like(ssum_o)
            ssq_o[...] = jnp.zeros_like(ssq_o)

        ssum_o[...] += jnp.sum(h1, axis=0, keepdims=True)
        ssq_o[...] += jnp.sum(h1 * h1, axis=0, keepdims=True)

    in_specs = ([pl.BlockSpec(memory_space=pltpu.SMEM)]
                + [pl.BlockSpec((_TN, 128), lambda i: (i, 0))] * (2 * C_in)
                + [pl.BlockSpec((din, _HID), lambda i: (0, 0)),
                   pl.BlockSpec((1, _HID), lambda i: (0, 0))])
    out_specs = [pl.BlockSpec((_TN, _HID), lambda i: (i, 0)),
                 pl.BlockSpec((1, _HID), lambda i: (0, 0)),
                 pl.BlockSpec((1, _HID), lambda i: (0, 0))]
    out_shape = [jax.ShapeDtypeStruct((_N, _HID), jnp.float32),
                 jax.ShapeDtypeStruct((1, _HID), jnp.float32),
                 jax.ShapeDtypeStruct((1, _HID), jnp.float32)]
    return pl.pallas_call(body, grid=grid, in_specs=in_specs,
                          out_specs=out_specs, out_shape=out_shape,
                          interpret=interpret)


@functools.lru_cache(maxsize=None)
def _mm2_kernel(interpret=False):
    grid = (_N // _TN,)

    def body(h1, ssum1, ssq1, g1, be1, w2, b2, h2_o, ssum2_o, ssq2_o):
        i = pl.program_id(0)
        m = ssum1[...] * (1.0 / _N)
        v = ssq1[...] * (1.0 / _N) - m * m
        sc = g1[...] * lax.rsqrt(v + 1e-5)
        sh = be1[...] - m * sc
        a = jnp.maximum(h1[...] * sc + sh, 0.0)
        h2 = _dot(a, w2[...]) + b2[...]
        h2_o[...] = h2

        @pl.when(i == 0)
        def _():
            ssum2_o[...] = jnp.zeros_like(ssum2_o)
            ssq2_o[...] = jnp.zeros_like(ssq2_o)

        ssum2_o[...] += jnp.sum(h2, axis=0, keepdims=True)
        ssq2_o[...] += jnp.sum(h2 * h2, axis=0, keepdims=True)

    in_specs = [pl.BlockSpec((_TN, _HID), lambda i: (i, 0)),
                pl.BlockSpec((1, _HID), lambda i: (0, 0)),
                pl.BlockSpec((1, _HID), lambda i: (0, 0)),
                pl.BlockSpec((1, _HID), lambda i: (0, 0)),
                pl.BlockSpec((1, _HID), lambda i: (0, 0)),
                pl.BlockSpec((_HID, _DOUT), lambda i: (0, 0)),
                pl.BlockSpec((1, _DOUT), lambda i: (0, 0))]
    out_specs = [pl.BlockSpec((_TN, _DOUT), lambda i: (i, 0)),
                 pl.BlockSpec((1, _DOUT), lambda i: (0, 0)),
                 pl.BlockSpec((1, _DOUT), lambda i: (0, 0))]
    out_shape = [jax.ShapeDtypeStruct((_N, _DOUT), jnp.float32),
                 jax.ShapeDtypeStruct((1, _DOUT), jnp.float32),
                 jax.ShapeDtypeStruct((1, _DOUT), jnp.float32)]
    return pl.pallas_call(body, grid=grid, in_specs=in_specs,
                          out_specs=out_specs, out_shape=out_shape,
                          interpret=interpret)


@functools.lru_cache(maxsize=None)
def _bn_relu_kernel(blocked, interpret=False):
    grid = (_N // _TN,)
    C_out = _DOUT // 128

    def body(h2, ssum2, ssq2, g, be, *outs):
        m = ssum2[...] * (1.0 / _N)
        v = ssq2[...] * (1.0 / _N) - m * m
        sc = g[...] * lax.rsqrt(v + 1e-5)
        sh = be[...] - m * sc
        y = jnp.maximum(h2[...] * sc + sh, 0.0)
        if blocked:
            for c in range(C_out):
                outs[c][...] = y[:, c * 128:(c + 1) * 128]
        else:
            outs[0][...] = y

    in_specs = [pl.BlockSpec((_TN, _DOUT), lambda i: (i, 0)),
                pl.BlockSpec((1, _DOUT), lambda i: (0, 0)),
                pl.BlockSpec((1, _DOUT), lambda i: (0, 0)),
                pl.BlockSpec((1, _DOUT), lambda i: (0, 0)),
                pl.BlockSpec((1, _DOUT), lambda i: (0, 0))]
    if blocked:
        out_specs = [pl.BlockSpec((_TN, 128), lambda i: (i, 0))] * C_out
        out_shape = [jax.ShapeDtypeStruct((_N, 128), jnp.float32)] * C_out
    else:
        out_specs = [pl.BlockSpec((_TN, _DOUT), lambda i: (i, 0))]
        out_shape = [jax.ShapeDtypeStruct((_N, _DOUT), jnp.float32)]
    return pl.pallas_call(body, grid=grid, in_specs=in_specs,
                          out_specs=out_specs, out_shape=out_shape,
                          interpret=interpret)


def kernel(x, edge_index, batch, params):
    src = edge_index[0]
    dst = edge_index[1]
    perm = jnp.argsort(dst, stable=True)
    src_s = src[perm]
    dst_s = dst[perm]
    npad_e = _EPAD - _E
    src_p = jnp.concatenate(
        [src_s, jnp.zeros((npad_e,), jnp.int32)]).reshape(_EROWS, 128)
    dst_p = jnp.concatenate(
        [dst_s, _N + (jnp.arange(npad_e, dtype=jnp.int32) % 240)]
    ).reshape(_EROWS, 128)
    zeros_pad = jnp.zeros((_NPAD, 128), jnp.float32)

    tables = [x[:, 0:128], x[:, 128:256]]
    out = None
    for l in range(4):
        C_in = len(tables)
        aggs = _seg_sum_kernel(C_in)(*tables, src_p, dst_p, zeros_pad)
        eps = (1.0 + params[f"eps_{l}"]).reshape(1, 1)
        h1, s1, q1 = _mm1_kernel(C_in)(
            eps, *aggs, *tables, params[f"W1_{l}"],
            params[f"b1_{l}"].reshape(1, -1))
        h2, s2, q2 = _mm2_kernel()(
            h1, s1, q1, params[f"g1_{l}"].reshape(1, -1),
            params[f"be1_{l}"].reshape(1, -1), params[f"W2_{l}"],
            params[f"b2_{l}"].reshape(1, -1))
        g = params[f"g_{l}"].reshape(1, -1)
        be = params[f"be_{l}"].reshape(1, -1)
        if l < 3:
            tables = list(_bn_relu_kernel(True)(h2, s2, q2, g, be))
        else:
            out = _bn_relu_kernel(False)(h2, s2, q2, g, be)[0]
    return out
